# Initial kernel scaffold; baseline (speedup 1.0000x reference)
import jax
import jax.numpy as jnp
from jax.experimental import pallas as pl


def kernel(x_author, x_topic, edge_index_a2t, edge_index_t2a, edge_label_index, params):
    raise NotImplementedError("write your pallas kernel here")


# speedup vs baseline: 2.3098x; 2.3098x over previous
import jax
import jax.numpy as jnp
from jax.experimental import pallas as pl

H = 128
NA = 10000
NT = 10000
L = 2


def _identity_kernel(x_ref, o_ref):
    o_ref[...] = x_ref[...]


def _identity(x):
    return pl.pallas_call(
        _identity_kernel,
        out_shape=jax.ShapeDtypeStruct(x.shape, x.dtype),
    )(x)


def kernel(x_author, x_topic, edge_index_a2t, edge_index_t2a, edge_label_index, params):
    p = params
    xa = jax.nn.relu(x_author @ p['W_in_author'] + p['b_in_author'])
    xt = jax.nn.relu(x_topic @ p['W_in_topic'] + p['b_in_topic'])
    sqrt_d = jnp.sqrt(jnp.float32(H))

    def agg(K, Q, V, row, col, n):
        k_e = K[row]
        v_e = V[row]
        s = jnp.sum(Q[col] * k_e, axis=-1)
        ex = jnp.exp(s)
        num = jax.ops.segment_sum(ex[:, None] * v_e, col, num_segments=n)
        den = jax.ops.segment_sum(ex, col, num_segments=n)
        return num / jnp.maximum(den, 1e-16)[:, None]

    for l in range(L):
        qkv = {}
        for t, rel in (('author', 'a2t'), ('topic', 't2a')):
            scale = p[f'l{l}_prel_{rel}'] / sqrt_d
            Wk_f = p[f'l{l}_Wk_{t}'] @ p[f'l{l}_arel_{rel}'] * scale
            bk_f = p[f'l{l}_bk_{t}'] @ p[f'l{l}_arel_{rel}'] * scale
            Wv_f = p[f'l{l}_Wv_{t}'] @ p[f'l{l}_mrel_{rel}']
            bv_f = p[f'l{l}_bv_{t}'] @ p[f'l{l}_mrel_{rel}']
            x = xa if t == 'author' else xt
            qkv[t] = (
                x @ p[f'l{l}_Wq_{t}'] + p[f'l{l}_bq_{t}'],
                x @ Wk_f + bk_f,
                x @ Wv_f + bv_f,
            )
        Qa, Ka, Va = qkv['author']
        Qt, Kt, Vt = qkv['topic']
        agg_t = agg(Ka, Qt, Va, edge_index_a2t[0], edge_index_a2t[1], NT)
        agg_a = agg(Kt, Qa, Vt, edge_index_t2a[0], edge_index_t2a[1], NA)
        out_a = jax.nn.gelu(agg_a, approximate=False) @ p[f'l{l}_Wa_author'] + p[f'l{l}_ba_author']
        out_t = jax.nn.gelu(agg_t, approximate=False) @ p[f'l{l}_Wa_topic'] + p[f'l{l}_ba_topic']
        sa = jax.nn.sigmoid(p[f'l{l}_skip_author'])
        st = jax.nn.sigmoid(p[f'l{l}_skip_topic'])
        xa = sa * out_a + (1.0 - sa) * xa
        xt = st * out_t + (1.0 - st) * xt

        def bn(x):
            mu = jnp.mean(x, axis=0)
            var = jnp.var(x, axis=0)
            return (x - mu) / jnp.sqrt(var + 1e-5) * p[f'l{l}_bn_gamma'] + p[f'l{l}_bn_beta']
        xa = bn(xa)
        xt = bn(xt)

    r, c = edge_label_index[0], edge_label_index[1]
    W1a = p['dec_W1'][:H]
    W1b = p['dec_W1'][H:]
    z = jax.nn.relu(xa[r] @ W1a + xa[c] @ W1b + p['dec_b1'])
    z = z @ p['dec_W2'] + p['dec_b2']
    return _identity(z.reshape(-1))


# speedup vs baseline: 4.3784x; 1.8956x over previous
import functools

import jax
import jax.numpy as jnp
from jax import lax
from jax.experimental import pallas as pl
from jax.experimental.pallas import tpu as pltpu
from jax.experimental.pallas import tpu_sc as plsc

H = 128
N = 10000
NE = 320000
NDEC = 100000

NC = 2
NS = 16
NW = NC * NS
LN = 16

AW = 144

EC = 128
ECA = 64
EPT_E = 10240
EV_E = NE // NW
NCH_E = EPT_E // ECA

EPT_D = 3200
NCH_D = EPT_D // EC
NPAD_D = NW * EPT_D

NROWP = 10240
RPT = NROWP // NS

_HI = lax.Precision.HIGHEST
_MESH = plsc.VectorSubcoreMesh(core_axis_name="c", subcore_axis_name="s")
_SC_PARAMS = pltpu.CompilerParams(use_tc_tiling_on_sc=False)


def _dot(a, b):
    return jnp.dot(a, b, preferred_element_type=jnp.float32, precision=_HI)


def _edge_agg_body(nrel, args):
    (*ops, out_hbm, rowv, colv, kbuf, qbuf, vbuf, obuf, acc_sh,
     sem0, sem1, sem2) = args
    cid = lax.axis_index("c")
    sid = lax.axis_index("s")
    wid = sid * NC + cid
    base_t = wid * EPT_E

    for r in range(nrel):
        k_hbm, q_hbm, v_hbm, row_hbm, col_hbm = ops[5 * r:5 * r + 5]

        @pl.loop(0, ECA)
        def _zero(i):
            for j in range(AW // LN):
                obuf[i, pl.ds(LN * j, LN)] = jnp.zeros((LN,), jnp.float32)

        for piece in range(RPT // ECA):
            pltpu.sync_copy(obuf, acc_sh.at[pl.ds(sid * RPT + piece * ECA, ECA)])
        plsc.subcore_barrier()

        @pl.loop(0, NCH_E)
        def _chunk(ck):
            gbase = base_t + ck * ECA
            pltpu.sync_copy(row_hbm.at[pl.ds(gbase, ECA)], rowv)
            pltpu.sync_copy(col_hbm.at[pl.ds(gbase, ECA)], colv)
            cpk = pltpu.async_copy(k_hbm.at[rowv], kbuf, sem0)
            cpq = pltpu.async_copy(q_hbm.at[colv], qbuf, sem1)
            cpv = pltpu.async_copy(v_hbm.at[rowv], vbuf, sem2)
            cpk.wait()
            cpq.wait()
            cpv.wait()
            nvalid = EV_E - ck * ECA

            lanes = lax.iota(jnp.int32, LN)

            @pl.loop(0, ECA)
            def _edge(e):
                acc = kbuf[e, pl.ds(0, LN)] * qbuf[e, pl.ds(0, LN)]
                for j in range(1, H // LN):
                    acc = acc + kbuf[e, pl.ds(LN * j, LN)] * qbuf[e, pl.ds(LN * j, LN)]
                for sh in (1, 2, 4, 8):
                    perm = jnp.bitwise_xor(lanes, sh)
                    acc = acc + acc.at[perm].get(mode="promise_in_bounds")
                vf = jnp.where(e < nvalid, jnp.float32(1.0), jnp.float32(0.0))
                ex = jnp.exp(acc) * vf
                obuf[e, pl.ds(H, LN)] = ex
                for j in range(H // LN):
                    obuf[e, pl.ds(LN * j, LN)] = vbuf[e, pl.ds(LN * j, LN)] * ex

            pltpu.sync_copy(obuf, acc_sh.at[colv], add=True)

        plsc.subcore_barrier()
        pltpu.sync_copy(acc_sh.at[pl.ds(sid * RPT, RPT)],
                        out_hbm.at[r, cid, pl.ds(sid * RPT, RPT)])


def _make_edge_agg(nrel):
    @functools.partial(
        pl.kernel,
        mesh=_MESH,
        compiler_params=_SC_PARAMS,
        out_type=jax.ShapeDtypeStruct((nrel, NC, NROWP, AW), jnp.float32),
        scratch_types=[
            pltpu.VMEM((ECA,), jnp.int32),
            pltpu.VMEM((ECA,), jnp.int32),
            pltpu.VMEM((ECA, H), jnp.float32),
            pltpu.VMEM((ECA, H), jnp.float32),
            pltpu.VMEM((ECA, H), jnp.float32),
            pltpu.VMEM((ECA, AW), jnp.float32),
            pltpu.VMEM_SHARED((NROWP, AW), jnp.float32),
            pltpu.SemaphoreType.DMA,
            pltpu.SemaphoreType.DMA,
            pltpu.SemaphoreType.DMA,
        ],
    )
    def _edge_agg(*args):
        _edge_agg_body(nrel, args)
    return _edge_agg


_edge_agg2 = _make_edge_agg(2)
_edge_agg1 = _make_edge_agg(1)


def _dec_gather_body(tab_hbm, r_hbm, c_hbm, gr_hbm, gc_hbm,
                     idxv, rbuf, sem0):
    cid = lax.axis_index("c")
    sid = lax.axis_index("s")
    wid = sid * NC + cid
    base_t = wid * EPT_D

    for idx_hbm, out_hbm in ((r_hbm, gr_hbm), (c_hbm, gc_hbm)):
        @pl.loop(0, NCH_D)
        def _chunk(ck):
            gbase = base_t + ck * EC
            pltpu.sync_copy(idx_hbm.at[pl.ds(gbase, EC)], idxv)
            pltpu.async_copy(tab_hbm.at[idxv], rbuf, sem0).wait()
            pltpu.sync_copy(rbuf, out_hbm.at[pl.ds(gbase, EC)])


@functools.partial(
    pl.kernel,
    mesh=_MESH,
    compiler_params=_SC_PARAMS,
    out_type=(jax.ShapeDtypeStruct((NPAD_D, H), jnp.float32),
              jax.ShapeDtypeStruct((NPAD_D, H), jnp.float32)),
    scratch_types=[
        pltpu.VMEM((EC,), jnp.int32),
        pltpu.VMEM((EC, H), jnp.float32),
        pltpu.SemaphoreType.DMA,
    ],
)
def _dec_gather(tab_hbm, r_hbm, c_hbm, gr_hbm, gc_hbm, *rest):
    _dec_gather_body(tab_hbm, r_hbm, c_hbm, gr_hbm, gc_hbm, *rest)


BROW = 2000


def _linear_relu_body(x_ref, w_ref, b_ref, o_ref):
    y = _dot(x_ref[...], w_ref[...]) + b_ref[...]
    o_ref[...] = jnp.maximum(y, 0.0)


def _linear_relu(x, w, b):
    return pl.pallas_call(
        _linear_relu_body,
        grid=(N // BROW,),
        in_specs=[
            pl.BlockSpec((BROW, H), lambda i: (i, 0)),
            pl.BlockSpec((H, H), lambda i: (0, 0)),
            pl.BlockSpec((1, H), lambda i: (0, 0)),
        ],
        out_specs=pl.BlockSpec((BROW, H), lambda i: (i, 0)),
        out_shape=jax.ShapeDtypeStruct((N, H), jnp.float32),
    )(x, w, b.reshape(1, H))


def _wfold_body(wk_ref, bk_ref, arel_ref, wv_ref, bv_ref, mrel_ref, prel_ref,
                kf_ref, bkf_ref, vf_ref, bvf_ref):
    scale = prel_ref[0, 0] / jnp.sqrt(jnp.float32(H))
    kf_ref[...] = _dot(wk_ref[...], arel_ref[...]) * scale
    bkf_ref[...] = _dot(bk_ref[...], arel_ref[...]) * scale
    vf_ref[...] = _dot(wv_ref[...], mrel_ref[...])
    bvf_ref[...] = _dot(bv_ref[...], mrel_ref[...])


def _wfold(wk, bk, arel, wv, bv, mrel, prel):
    return pl.pallas_call(
        _wfold_body,
        out_shape=(jax.ShapeDtypeStruct((H, H), jnp.float32),
                   jax.ShapeDtypeStruct((1, H), jnp.float32),
                   jax.ShapeDtypeStruct((H, H), jnp.float32),
                   jax.ShapeDtypeStruct((1, H), jnp.float32)),
    )(wk, bk.reshape(1, H), arel, wv, bv.reshape(1, H), mrel,
      prel.reshape(1, 1))


def _qkv_body(x_ref, wq_ref, bq_ref, kf_ref, bkf_ref, vf_ref, bvf_ref,
              q_ref, k_ref, v_ref):
    x = x_ref[...]
    q_ref[...] = _dot(x, wq_ref[...]) + bq_ref[...]
    k_ref[...] = _dot(x, kf_ref[...]) + bkf_ref[...]
    v_ref[...] = _dot(x, vf_ref[...]) + bvf_ref[...]


def _qkv(x, wq, bq, kf, bkf, vf, bvf):
    full = lambda i: (0, 0)
    blk = lambda i: (i, 0)
    return pl.pallas_call(
        _qkv_body,
        grid=(N // BROW,),
        in_specs=[
            pl.BlockSpec((BROW, H), blk),
            pl.BlockSpec((H, H), full),
            pl.BlockSpec((1, H), full),
            pl.BlockSpec((H, H), full),
            pl.BlockSpec((1, H), full),
            pl.BlockSpec((H, H), full),
            pl.BlockSpec((1, H), full),
        ],
        out_specs=(pl.BlockSpec((BROW, H), blk),) * 3,
        out_shape=(jax.ShapeDtypeStruct((N, H), jnp.float32),) * 3,
    )(x, wq, bq.reshape(1, H), kf, bkf, vf, bvf)


def _post_a_body(acc_ref, xp_ref, wa_ref, ba_ref, skip_ref, y_ref, st_ref):
    i = pl.program_id(0)
    acc = acc_ref[0] + acc_ref[1]
    den = acc[:, H:H + 1]
    agg = acc[:, :H] / jnp.maximum(den, 1e-16)
    g = 0.5 * agg * (1.0 + lax.erf(agg / jnp.sqrt(jnp.float32(2.0))))
    out = _dot(g, wa_ref[...]) + ba_ref[...]
    s = jax.nn.sigmoid(skip_ref[0, 0])
    y = s * out + (1.0 - s) * xp_ref[...]
    y_ref[...] = y

    @pl.when(i == 0)
    def _():
        st_ref[...] = jnp.zeros_like(st_ref)

    st_ref[...] += jnp.concatenate(
        [jnp.sum(y, axis=0, keepdims=True),
         jnp.sum(y * y, axis=0, keepdims=True)], axis=0)


def _post_a(acc2, x_prev, wa, ba, skip):
    return pl.pallas_call(
        _post_a_body,
        grid=(N // BROW,),
        in_specs=[
            pl.BlockSpec((NC, BROW, AW), lambda i: (0, i, 0)),
            pl.BlockSpec((BROW, H), lambda i: (i, 0)),
            pl.BlockSpec((H, H), lambda i: (0, 0)),
            pl.BlockSpec((1, H), lambda i: (0, 0)),
            pl.BlockSpec((1, 1), lambda i: (0, 0)),
        ],
        out_specs=(pl.BlockSpec((BROW, H), lambda i: (i, 0)),
                   pl.BlockSpec((2, H), lambda i: (0, 0))),
        out_shape=(jax.ShapeDtypeStruct((N, H), jnp.float32),
                   jax.ShapeDtypeStruct((2, H), jnp.float32)),
    )(acc2, x_prev, wa, ba.reshape(1, H), skip.reshape(1, 1))


def _post_b_body(y_ref, st_ref, g_ref, b_ref, o_ref):
    mu = st_ref[0:1] / jnp.float32(N)
    var = st_ref[1:2] / jnp.float32(N) - mu * mu
    o_ref[...] = (y_ref[...] - mu) / jnp.sqrt(var + 1e-5) * g_ref[...] + b_ref[...]


def _post_b(y, st, gamma, beta):
    return pl.pallas_call(
        _post_b_body,
        grid=(N // BROW,),
        in_specs=[
            pl.BlockSpec((BROW, H), lambda i: (i, 0)),
            pl.BlockSpec((2, H), lambda i: (0, 0)),
            pl.BlockSpec((1, H), lambda i: (0, 0)),
            pl.BlockSpec((1, H), lambda i: (0, 0)),
        ],
        out_specs=pl.BlockSpec((BROW, H), lambda i: (i, 0)),
        out_shape=jax.ShapeDtypeStruct((N, H), jnp.float32),
    )(y, st, gamma.reshape(1, H), beta.reshape(1, H))


BDEC = 2048


def _dec_body(gr_ref, gc_ref, w1a_ref, w1b_ref, b1_ref, w2_ref, b2_ref, o_ref):
    z1 = _dot(gr_ref[...], w1a_ref[...]) + _dot(gc_ref[...], w1b_ref[...])
    z1 = jnp.maximum(z1 + b1_ref[...], 0.0)
    o_ref[...] = jnp.sum(z1 * w2_ref[...], axis=1) + b2_ref[0, 0]


def _decoder(gr, gc, w1a, w1b, b1, w2r, b2):
    full = lambda i: (0, 0)
    return pl.pallas_call(
        _dec_body,
        grid=(NPAD_D // BDEC,),
        in_specs=[
            pl.BlockSpec((BDEC, H), lambda i: (i, 0)),
            pl.BlockSpec((BDEC, H), lambda i: (i, 0)),
            pl.BlockSpec((H, H), full),
            pl.BlockSpec((H, H), full),
            pl.BlockSpec((1, H), full),
            pl.BlockSpec((1, H), full),
            pl.BlockSpec((1, 1), full),
        ],
        out_specs=pl.BlockSpec((BDEC,), lambda i: (i,)),
        out_shape=jax.ShapeDtypeStruct((NPAD_D,), jnp.float32),
    )(gr, gc, w1a, w1b, b1.reshape(1, H), w2r, b2.reshape(1, 1))


def _pad_tiles(idx, valid_per_tile, padded_per_tile):
    x = idx.astype(jnp.int32).reshape(NW, valid_per_tile)
    x = jnp.pad(x, ((0, 0), (0, padded_per_tile - valid_per_tile)))
    return x.reshape(-1)


def kernel(x_author, x_topic, edge_index_a2t, edge_index_t2a, edge_label_index, params):
    p = params

    rows_a2t = _pad_tiles(edge_index_a2t[0], EV_E, EPT_E)
    cols_a2t = _pad_tiles(edge_index_a2t[1], EV_E, EPT_E)
    rows_t2a = _pad_tiles(edge_index_t2a[0], EV_E, EPT_E)
    cols_t2a = _pad_tiles(edge_index_t2a[1], EV_E, EPT_E)
    dec_r = _pad_tiles(edge_label_index[0], NDEC // NW, EPT_D)
    dec_c = _pad_tiles(edge_label_index[1], NDEC // NW, EPT_D)

    xa = _linear_relu(x_author, p['W_in_author'], p['b_in_author'])
    xt = _linear_relu(x_topic, p['W_in_topic'], p['b_in_topic'])

    for l in range(2):
        last = l == 1
        kft, bkft, vft, bvft = _wfold(
            p[f'l{l}_Wk_topic'], p[f'l{l}_bk_topic'], p[f'l{l}_arel_t2a'],
            p[f'l{l}_Wv_topic'], p[f'l{l}_bv_topic'], p[f'l{l}_mrel_t2a'],
            p[f'l{l}_prel_t2a'])
        qa, ka, va = _qkv(xa, p[f'l{l}_Wq_author'], p[f'l{l}_bq_author'],
                          *( _wfold(
                              p[f'l{l}_Wk_author'], p[f'l{l}_bk_author'],
                              p[f'l{l}_arel_a2t'],
                              p[f'l{l}_Wv_author'], p[f'l{l}_bv_author'],
                              p[f'l{l}_mrel_a2t'], p[f'l{l}_prel_a2t'])))
        qt, kt, vt = _qkv(xt, p[f'l{l}_Wq_topic'], p[f'l{l}_bq_topic'],
                          kft, bkft, vft, bvft)

        if last:
            acc = _edge_agg1(kt, qa, vt, rows_t2a, cols_t2a)
            acc_a = acc[0]
        else:
            acc = _edge_agg2(ka, qt, va, rows_a2t, cols_a2t,
                             kt, qa, vt, rows_t2a, cols_t2a)
            acc_t, acc_a = acc[0], acc[1]

        ya, sta = _post_a(acc_a, xa, p[f'l{l}_Wa_author'], p[f'l{l}_ba_author'],
                          p[f'l{l}_skip_author'])
        xa = _post_b(ya, sta, p[f'l{l}_bn_gamma'], p[f'l{l}_bn_beta'])
        if not last:
            yt, stt = _post_a(acc_t, xt, p[f'l{l}_Wa_topic'], p[f'l{l}_ba_topic'],
                              p[f'l{l}_skip_topic'])
            xt = _post_b(yt, stt, p[f'l{l}_bn_gamma'], p[f'l{l}_bn_beta'])

    gr, gc = _dec_gather(xa, dec_r, dec_c)
    z = _decoder(gr, gc, p['dec_W1'][:H], p['dec_W1'][H:], p['dec_b1'],
                 p['dec_W2'].reshape(1, H), p['dec_b2'])
    return z.reshape(NW, EPT_D)[:, :NDEC // NW].reshape(-1)


# speedup vs baseline: 4.6498x; 1.0620x over previous
import functools

import jax
import jax.numpy as jnp
from jax import lax
from jax.experimental import pallas as pl
from jax.experimental.pallas import tpu as pltpu
from jax.experimental.pallas import tpu_sc as plsc

H = 128
N = 10000
NE = 320000
NDEC = 100000

NC = 2
NS = 16
NW = NC * NS
LN = 16

AW = 144

EC = 128
ECA = 32
EPT_E = 10240
EV_E = NE // NW
NCH_E = EPT_E // ECA

EPT_D = 3200
NCH_D = EPT_D // EC
NPAD_D = NW * EPT_D

NROWP = 10240
RPT = NROWP // NS

_HI = lax.Precision.HIGHEST
_MESH = plsc.VectorSubcoreMesh(core_axis_name="c", subcore_axis_name="s")
_SC_PARAMS = pltpu.CompilerParams(use_tc_tiling_on_sc=False)


def _dot(a, b):
    return jnp.dot(a, b, preferred_element_type=jnp.float32, precision=_HI)


def _edge_agg_body(nrel, args):
    (*ops, out_hbm, rv0, cv0, rv1, cv1, k0, q0, v0, k1, q1, v1, obuf,
     acc_sh, sem0, sem1) = args
    rv = (rv0, rv1)
    cv = (cv0, cv1)
    kb = (k0, k1)
    qb = (q0, q1)
    vb = (v0, v1)
    sems = (sem0, sem1)
    cid = lax.axis_index("c")
    sid = lax.axis_index("s")
    wid = sid * NC + cid
    base_t = wid * EPT_E
    lanes = lax.iota(jnp.int32, LN)

    for r in range(nrel):
        k_hbm, q_hbm, v_hbm, row_hbm, col_hbm = ops[5 * r:5 * r + 5]

        @pl.loop(0, ECA)
        def _zero(i):
            for j in range(AW // LN):
                obuf[i, pl.ds(LN * j, LN)] = jnp.zeros((LN,), jnp.float32)

        for piece in range(RPT // ECA):
            pltpu.sync_copy(obuf, acc_sh.at[pl.ds(sid * RPT + piece * ECA, ECA)])
        plsc.subcore_barrier()

        def _load_idx(ck, b):
            gbase = base_t + ck * ECA
            pltpu.sync_copy(row_hbm.at[pl.ds(gbase, ECA)], rv[b])
            pltpu.sync_copy(col_hbm.at[pl.ds(gbase, ECA)], cv[b])

        def _issue(b):
            return (pltpu.async_copy(k_hbm.at[rv[b]], kb[b], sems[b]),
                    pltpu.async_copy(q_hbm.at[cv[b]], qb[b], sems[b]),
                    pltpu.async_copy(v_hbm.at[rv[b]], vb[b], sems[b]))

        descs = []
        for b in range(2):
            _load_idx(b, b)
            descs.append(_issue(b))

        @pl.loop(0, NCH_E, step=2)
        def _chunk(g):
            for b in range(2):
                ck = g + b
                for d in descs[b]:
                    d.wait()
                nvalid = EV_E - ck * ECA

                @pl.loop(0, ECA)
                def _edge(e):
                    acc = kb[b][e, pl.ds(0, LN)] * qb[b][e, pl.ds(0, LN)]
                    for j in range(1, H // LN):
                        acc = acc + (kb[b][e, pl.ds(LN * j, LN)]
                                     * qb[b][e, pl.ds(LN * j, LN)])
                    for sh in (1, 2, 4, 8):
                        perm = jnp.bitwise_xor(lanes, sh)
                        acc = acc + acc.at[perm].get(mode="promise_in_bounds")
                    vf = jnp.where(e < nvalid, jnp.float32(1.0), jnp.float32(0.0))
                    ex = jnp.exp(acc) * vf
                    obuf[e, pl.ds(H, LN)] = ex
                    for j in range(H // LN):
                        obuf[e, pl.ds(LN * j, LN)] = vb[b][e, pl.ds(LN * j, LN)] * ex

                pltpu.sync_copy(obuf, acc_sh.at[cv[b]], add=True)

                @pl.when(ck + 2 < NCH_E)
                def _pf():
                    _load_idx(ck + 2, b)
                    _issue(b)

        plsc.subcore_barrier()
        pltpu.sync_copy(acc_sh.at[pl.ds(sid * RPT, RPT)],
                        out_hbm.at[r, cid, pl.ds(sid * RPT, RPT)])


def _make_edge_agg(nrel):
    @functools.partial(
        pl.kernel,
        mesh=_MESH,
        compiler_params=_SC_PARAMS,
        out_type=jax.ShapeDtypeStruct((nrel, NC, NROWP, AW), jnp.float32),
        scratch_types=[
            pltpu.VMEM((ECA,), jnp.int32),
            pltpu.VMEM((ECA,), jnp.int32),
            pltpu.VMEM((ECA,), jnp.int32),
            pltpu.VMEM((ECA,), jnp.int32),
            pltpu.VMEM((ECA, H), jnp.float32),
            pltpu.VMEM((ECA, H), jnp.float32),
            pltpu.VMEM((ECA, H), jnp.float32),
            pltpu.VMEM((ECA, H), jnp.float32),
            pltpu.VMEM((ECA, H), jnp.float32),
            pltpu.VMEM((ECA, H), jnp.float32),
            pltpu.VMEM((ECA, AW), jnp.float32),
            pltpu.VMEM_SHARED((NROWP, AW), jnp.float32),
            pltpu.SemaphoreType.DMA,
            pltpu.SemaphoreType.DMA,
        ],
    )
    def _edge_agg(*args):
        _edge_agg_body(nrel, args)
    return _edge_agg


_edge_agg2 = _make_edge_agg(2)
_edge_agg1 = _make_edge_agg(1)


def _dec_gather_body(tab_hbm, r_hbm, c_hbm, gr_hbm, gc_hbm,
                     idxv, rbuf, sem0):
    cid = lax.axis_index("c")
    sid = lax.axis_index("s")
    wid = sid * NC + cid
    base_t = wid * EPT_D

    for idx_hbm, out_hbm in ((r_hbm, gr_hbm), (c_hbm, gc_hbm)):
        @pl.loop(0, NCH_D)
        def _chunk(ck):
            gbase = base_t + ck * EC
            pltpu.sync_copy(idx_hbm.at[pl.ds(gbase, EC)], idxv)
            pltpu.async_copy(tab_hbm.at[idxv], rbuf, sem0).wait()
            pltpu.sync_copy(rbuf, out_hbm.at[pl.ds(gbase, EC)])


@functools.partial(
    pl.kernel,
    mesh=_MESH,
    compiler_params=_SC_PARAMS,
    out_type=(jax.ShapeDtypeStruct((NPAD_D, H), jnp.float32),
              jax.ShapeDtypeStruct((NPAD_D, H), jnp.float32)),
    scratch_types=[
        pltpu.VMEM((EC,), jnp.int32),
        pltpu.VMEM((EC, H), jnp.float32),
        pltpu.SemaphoreType.DMA,
    ],
)
def _dec_gather(tab_hbm, r_hbm, c_hbm, gr_hbm, gc_hbm, *rest):
    _dec_gather_body(tab_hbm, r_hbm, c_hbm, gr_hbm, gc_hbm, *rest)


BROW = 2000


def _linear_relu_body(x_ref, w_ref, b_ref, o_ref):
    y = _dot(x_ref[...], w_ref[...]) + b_ref[...]
    o_ref[...] = jnp.maximum(y, 0.0)


def _linear_relu(x, w, b):
    return pl.pallas_call(
        _linear_relu_body,
        grid=(N // BROW,),
        in_specs=[
            pl.BlockSpec((BROW, H), lambda i: (i, 0)),
            pl.BlockSpec((H, H), lambda i: (0, 0)),
            pl.BlockSpec((1, H), lambda i: (0, 0)),
        ],
        out_specs=pl.BlockSpec((BROW, H), lambda i: (i, 0)),
        out_shape=jax.ShapeDtypeStruct((N, H), jnp.float32),
    )(x, w, b.reshape(1, H))


def _wfold_body(wk_ref, bk_ref, arel_ref, wv_ref, bv_ref, mrel_ref, prel_ref,
                kf_ref, bkf_ref, vf_ref, bvf_ref):
    scale = prel_ref[0, 0] / jnp.sqrt(jnp.float32(H))
    kf_ref[...] = _dot(wk_ref[...], arel_ref[...]) * scale
    bkf_ref[...] = _dot(bk_ref[...], arel_ref[...]) * scale
    vf_ref[...] = _dot(wv_ref[...], mrel_ref[...])
    bvf_ref[...] = _dot(bv_ref[...], mrel_ref[...])


def _wfold(wk, bk, arel, wv, bv, mrel, prel):
    return pl.pallas_call(
        _wfold_body,
        out_shape=(jax.ShapeDtypeStruct((H, H), jnp.float32),
                   jax.ShapeDtypeStruct((1, H), jnp.float32),
                   jax.ShapeDtypeStruct((H, H), jnp.float32),
                   jax.ShapeDtypeStruct((1, H), jnp.float32)),
    )(wk, bk.reshape(1, H), arel, wv, bv.reshape(1, H), mrel,
      prel.reshape(1, 1))


def _qkv_body(x_ref, wq_ref, bq_ref, kf_ref, bkf_ref, vf_ref, bvf_ref,
              q_ref, k_ref, v_ref):
    x = x_ref[...]
    q_ref[...] = _dot(x, wq_ref[...]) + bq_ref[...]
    k_ref[...] = _dot(x, kf_ref[...]) + bkf_ref[...]
    v_ref[...] = _dot(x, vf_ref[...]) + bvf_ref[...]


def _qkv(x, wq, bq, kf, bkf, vf, bvf):
    full = lambda i: (0, 0)
    blk = lambda i: (i, 0)
    return pl.pallas_call(
        _qkv_body,
        grid=(N // BROW,),
        in_specs=[
            pl.BlockSpec((BROW, H), blk),
            pl.BlockSpec((H, H), full),
            pl.BlockSpec((1, H), full),
            pl.BlockSpec((H, H), full),
            pl.BlockSpec((1, H), full),
            pl.BlockSpec((H, H), full),
            pl.BlockSpec((1, H), full),
        ],
        out_specs=(pl.BlockSpec((BROW, H), blk),) * 3,
        out_shape=(jax.ShapeDtypeStruct((N, H), jnp.float32),) * 3,
    )(x, wq, bq.reshape(1, H), kf, bkf, vf, bvf)


def _post_a_body(acc_ref, xp_ref, wa_ref, ba_ref, skip_ref, y_ref, st_ref):
    i = pl.program_id(0)
    acc = acc_ref[0] + acc_ref[1]
    den = acc[:, H:H + 1]
    agg = acc[:, :H] / jnp.maximum(den, 1e-16)
    g = 0.5 * agg * (1.0 + lax.erf(agg / jnp.sqrt(jnp.float32(2.0))))
    out = _dot(g, wa_ref[...]) + ba_ref[...]
    s = jax.nn.sigmoid(skip_ref[0, 0])
    y = s * out + (1.0 - s) * xp_ref[...]
    y_ref[...] = y

    @pl.when(i == 0)
    def _():
        st_ref[...] = jnp.zeros_like(st_ref)

    st_ref[...] += jnp.concatenate(
        [jnp.sum(y, axis=0, keepdims=True),
         jnp.sum(y * y, axis=0, keepdims=True)], axis=0)


def _post_a(acc2, x_prev, wa, ba, skip):
    return pl.pallas_call(
        _post_a_body,
        grid=(N // BROW,),
        in_specs=[
            pl.BlockSpec((NC, BROW, AW), lambda i: (0, i, 0)),
            pl.BlockSpec((BROW, H), lambda i: (i, 0)),
            pl.BlockSpec((H, H), lambda i: (0, 0)),
            pl.BlockSpec((1, H), lambda i: (0, 0)),
            pl.BlockSpec((1, 1), lambda i: (0, 0)),
        ],
        out_specs=(pl.BlockSpec((BROW, H), lambda i: (i, 0)),
                   pl.BlockSpec((2, H), lambda i: (0, 0))),
        out_shape=(jax.ShapeDtypeStruct((N, H), jnp.float32),
                   jax.ShapeDtypeStruct((2, H), jnp.float32)),
    )(acc2, x_prev, wa, ba.reshape(1, H), skip.reshape(1, 1))


def _post_b_body(y_ref, st_ref, g_ref, b_ref, o_ref):
    mu = st_ref[0:1] / jnp.float32(N)
    var = st_ref[1:2] / jnp.float32(N) - mu * mu
    o_ref[...] = (y_ref[...] - mu) / jnp.sqrt(var + 1e-5) * g_ref[...] + b_ref[...]


def _post_b(y, st, gamma, beta):
    return pl.pallas_call(
        _post_b_body,
        grid=(N // BROW,),
        in_specs=[
            pl.BlockSpec((BROW, H), lambda i: (i, 0)),
            pl.BlockSpec((2, H), lambda i: (0, 0)),
            pl.BlockSpec((1, H), lambda i: (0, 0)),
            pl.BlockSpec((1, H), lambda i: (0, 0)),
        ],
        out_specs=pl.BlockSpec((BROW, H), lambda i: (i, 0)),
        out_shape=jax.ShapeDtypeStruct((N, H), jnp.float32),
    )(y, st, gamma.reshape(1, H), beta.reshape(1, H))


BDEC = 2048


def _dec_body(gr_ref, gc_ref, w1a_ref, w1b_ref, b1_ref, w2_ref, b2_ref, o_ref):
    z1 = _dot(gr_ref[...], w1a_ref[...]) + _dot(gc_ref[...], w1b_ref[...])
    z1 = jnp.maximum(z1 + b1_ref[...], 0.0)
    o_ref[...] = jnp.sum(z1 * w2_ref[...], axis=1) + b2_ref[0, 0]


def _decoder(gr, gc, w1a, w1b, b1, w2r, b2):
    full = lambda i: (0, 0)
    return pl.pallas_call(
        _dec_body,
        grid=(NPAD_D // BDEC,),
        in_specs=[
            pl.BlockSpec((BDEC, H), lambda i: (i, 0)),
            pl.BlockSpec((BDEC, H), lambda i: (i, 0)),
            pl.BlockSpec((H, H), full),
            pl.BlockSpec((H, H), full),
            pl.BlockSpec((1, H), full),
            pl.BlockSpec((1, H), full),
            pl.BlockSpec((1, 1), full),
        ],
        out_specs=pl.BlockSpec((BDEC,), lambda i: (i,)),
        out_shape=jax.ShapeDtypeStruct((NPAD_D,), jnp.float32),
    )(gr, gc, w1a, w1b, b1.reshape(1, H), w2r, b2.reshape(1, 1))


def _pad_tiles(idx, valid_per_tile, padded_per_tile):
    x = idx.astype(jnp.int32).reshape(NW, valid_per_tile)
    x = jnp.pad(x, ((0, 0), (0, padded_per_tile - valid_per_tile)))
    return x.reshape(-1)


def kernel(x_author, x_topic, edge_index_a2t, edge_index_t2a, edge_label_index, params):
    p = params

    rows_a2t = _pad_tiles(edge_index_a2t[0], EV_E, EPT_E)
    cols_a2t = _pad_tiles(edge_index_a2t[1], EV_E, EPT_E)
    rows_t2a = _pad_tiles(edge_index_t2a[0], EV_E, EPT_E)
    cols_t2a = _pad_tiles(edge_index_t2a[1], EV_E, EPT_E)
    dec_r = _pad_tiles(edge_label_index[0], NDEC // NW, EPT_D)
    dec_c = _pad_tiles(edge_label_index[1], NDEC // NW, EPT_D)

    xa = _linear_relu(x_author, p['W_in_author'], p['b_in_author'])
    xt = _linear_relu(x_topic, p['W_in_topic'], p['b_in_topic'])

    for l in range(2):
        last = l == 1
        kft, bkft, vft, bvft = _wfold(
            p[f'l{l}_Wk_topic'], p[f'l{l}_bk_topic'], p[f'l{l}_arel_t2a'],
            p[f'l{l}_Wv_topic'], p[f'l{l}_bv_topic'], p[f'l{l}_mrel_t2a'],
            p[f'l{l}_prel_t2a'])
        qa, ka, va = _qkv(xa, p[f'l{l}_Wq_author'], p[f'l{l}_bq_author'],
                          *( _wfold(
                              p[f'l{l}_Wk_author'], p[f'l{l}_bk_author'],
                              p[f'l{l}_arel_a2t'],
                              p[f'l{l}_Wv_author'], p[f'l{l}_bv_author'],
                              p[f'l{l}_mrel_a2t'], p[f'l{l}_prel_a2t'])))
        qt, kt, vt = _qkv(xt, p[f'l{l}_Wq_topic'], p[f'l{l}_bq_topic'],
                          kft, bkft, vft, bvft)

        if last:
            acc = _edge_agg1(kt, qa, vt, rows_t2a, cols_t2a)
            acc_a = acc[0]
        else:
            acc = _edge_agg2(ka, qt, va, rows_a2t, cols_a2t,
                             kt, qa, vt, rows_t2a, cols_t2a)
            acc_t, acc_a = acc[0], acc[1]

        ya, sta = _post_a(acc_a, xa, p[f'l{l}_Wa_author'], p[f'l{l}_ba_author'],
                          p[f'l{l}_skip_author'])
        xa = _post_b(ya, sta, p[f'l{l}_bn_gamma'], p[f'l{l}_bn_beta'])
        if not last:
            yt, stt = _post_a(acc_t, xt, p[f'l{l}_Wa_topic'], p[f'l{l}_ba_topic'],
                              p[f'l{l}_skip_topic'])
            xt = _post_b(yt, stt, p[f'l{l}_bn_gamma'], p[f'l{l}_bn_beta'])

    gr, gc = _dec_gather(xa, dec_r, dec_c)
    z = _decoder(gr, gc, p['dec_W1'][:H], p['dec_W1'][H:], p['dec_b1'],
                 p['dec_W2'].reshape(1, H), p['dec_b2'])
    return z.reshape(NW, EPT_D)[:, :NDEC // NW].reshape(-1)


# speedup vs baseline: 4.6784x; 1.0062x over previous
import functools

import jax
import jax.numpy as jnp
from jax import lax
from jax.experimental import pallas as pl
from jax.experimental.pallas import tpu as pltpu
from jax.experimental.pallas import tpu_sc as plsc

H = 128
N = 10000
NE = 320000
NDEC = 100000

NC = 2
NS = 16
NW = NC * NS
LN = 16

AW = 144

EC = 128
ECA = 32
EPT_E = 10240
EV_E = NE // NW
NCH_E = EPT_E // ECA

EPT_D = 3200
NCH_D = EPT_D // EC
NPAD_D = NW * EPT_D

NROWP = 10240
RPT = NROWP // NS

_REV = (0, 8, 4, 12, 2, 10, 6, 14, 1, 9, 5, 13, 3, 11, 7, 15)

_HI = lax.Precision.HIGHEST
_MESH = plsc.VectorSubcoreMesh(core_axis_name="c", subcore_axis_name="s")
_SC_PARAMS = pltpu.CompilerParams(use_tc_tiling_on_sc=False)


def _dot(a, b):
    return jnp.dot(a, b, preferred_element_type=jnp.float32, precision=_HI)


def _edge_agg_body(nrel, args):
    (*ops, out_hbm, rv0, cv0, rv1, cv1, sv0, k0, q0, v0, k1, q1, v1,
     ob0, acc_sh, sem0, sem1) = args
    rv = (rv0, rv1)
    cv = (cv0, cv1)
    kb = (k0, k1)
    qb = (q0, q1)
    vb = (v0, v1)
    sems = (sem0, sem1)
    cid = lax.axis_index("c")
    sid = lax.axis_index("s")
    wid = sid * NC + cid
    base_t = wid * EPT_E
    lanes = lax.iota(jnp.int32, LN)
    rev = (jnp.left_shift(jnp.bitwise_and(lanes, 1), 3)
           | jnp.left_shift(jnp.bitwise_and(lanes, 2), 1)
           | jnp.right_shift(jnp.bitwise_and(lanes, 4), 1)
           | jnp.right_shift(jnp.bitwise_and(lanes, 8), 3))

    for r in range(nrel):
        k_hbm, q_hbm, v_hbm, row_hbm, col_hbm = ops[5 * r:5 * r + 5]

        @pl.loop(0, 2 * ECA)
        def _zero(i):
            for j in range(AW // LN):
                ob0[i, pl.ds(LN * j, LN)] = jnp.zeros((LN,), jnp.float32)

        for piece in range(RPT // (2 * ECA)):
            pltpu.sync_copy(
                ob0, acc_sh.at[pl.ds(sid * RPT + piece * 2 * ECA, 2 * ECA)])
        plsc.subcore_barrier()

        def _load_idx(ck, b):
            gbase = base_t + ck * ECA
            pltpu.sync_copy(row_hbm.at[pl.ds(gbase, ECA)], rv[b])
            pltpu.sync_copy(col_hbm.at[pl.ds(gbase, ECA)], cv[b])

        def _issue(b):
            return (pltpu.async_copy(k_hbm.at[rv[b]], kb[b], sems[b]),
                    pltpu.async_copy(q_hbm.at[cv[b]], qb[b], sems[b]),
                    pltpu.async_copy(v_hbm.at[rv[b]], vb[b], sems[b]))

        descs = []
        for b in range(2):
            _load_idx(b, b)
            descs.append(_issue(b))

        @pl.loop(0, NCH_E, step=2)
        def _chunk(g):
            for b in range(2):
                ck = g + b
                boff = b * ECA
                for d in descs[b]:
                    d.wait()
                for piece in range(ECA // LN):
                    sv0[pl.ds(boff + piece * LN, LN)] = cv[b][pl.ds(piece * LN, LN)]
                nvalid = EV_E - ck * ECA

                @pl.loop(0, ECA)
                def _edge(e):
                    acc = kb[b][e, pl.ds(0, LN)] * qb[b][e, pl.ds(0, LN)]
                    for j in range(1, H // LN):
                        acc = acc + (kb[b][e, pl.ds(LN * j, LN)]
                                     * qb[b][e, pl.ds(LN * j, LN)])
                    for sh in (1, 2, 4, 8):
                        perm = jnp.bitwise_xor(lanes, sh)
                        acc = acc + acc.at[perm].get(mode="promise_in_bounds")
                    vf = jnp.where(e < nvalid, jnp.float32(1.0), jnp.float32(0.0))
                    ex = jnp.exp(acc) * vf
                    ob0[boff + e, pl.ds(H, LN)] = ex
                    for j in range(H // LN):
                        ob0[boff + e, pl.ds(LN * j, LN)] = (
                            vb[b][e, pl.ds(LN * j, LN)] * ex)

                @pl.when(ck + 2 < NCH_E)
                def _pf():
                    _load_idx(ck + 2, b)
                    _issue(b)

            pltpu.sync_copy(ob0, acc_sh.at[sv0], add=True)

        plsc.subcore_barrier()
        pltpu.sync_copy(acc_sh.at[pl.ds(sid * RPT, RPT)],
                        out_hbm.at[r, cid, pl.ds(sid * RPT, RPT)])


def _make_edge_agg(nrel):
    @functools.partial(
        pl.kernel,
        mesh=_MESH,
        compiler_params=_SC_PARAMS,
        out_type=jax.ShapeDtypeStruct((nrel, NC, NROWP, AW), jnp.float32),
        scratch_types=[
            pltpu.VMEM((ECA,), jnp.int32),
            pltpu.VMEM((ECA,), jnp.int32),
            pltpu.VMEM((ECA,), jnp.int32),
            pltpu.VMEM((ECA,), jnp.int32),
            pltpu.VMEM((2 * ECA,), jnp.int32),
            pltpu.VMEM((ECA, H), jnp.float32),
            pltpu.VMEM((ECA, H), jnp.float32),
            pltpu.VMEM((ECA, H), jnp.float32),
            pltpu.VMEM((ECA, H), jnp.float32),
            pltpu.VMEM((ECA, H), jnp.float32),
            pltpu.VMEM((ECA, H), jnp.float32),
            pltpu.VMEM((2 * ECA, AW), jnp.float32),
            pltpu.VMEM_SHARED((NROWP, AW), jnp.float32),
            pltpu.SemaphoreType.DMA,
            pltpu.SemaphoreType.DMA,
        ],
    )
    def _edge_agg(*args):
        _edge_agg_body(nrel, args)
    return _edge_agg


_edge_agg2 = _make_edge_agg(2)
_edge_agg1 = _make_edge_agg(1)


def _dec_gather_body(tab_hbm, r_hbm, c_hbm, gr_hbm, gc_hbm,
                     idxv, rbuf, sem0):
    cid = lax.axis_index("c")
    sid = lax.axis_index("s")
    wid = sid * NC + cid
    base_t = wid * EPT_D

    for idx_hbm, out_hbm in ((r_hbm, gr_hbm), (c_hbm, gc_hbm)):
        @pl.loop(0, NCH_D)
        def _chunk(ck):
            gbase = base_t + ck * EC
            pltpu.sync_copy(idx_hbm.at[pl.ds(gbase, EC)], idxv)
            pltpu.async_copy(tab_hbm.at[idxv], rbuf, sem0).wait()
            pltpu.sync_copy(rbuf, out_hbm.at[pl.ds(gbase, EC)])


@functools.partial(
    pl.kernel,
    mesh=_MESH,
    compiler_params=_SC_PARAMS,
    out_type=(jax.ShapeDtypeStruct((NPAD_D, H), jnp.float32),
              jax.ShapeDtypeStruct((NPAD_D, H), jnp.float32)),
    scratch_types=[
        pltpu.VMEM((EC,), jnp.int32),
        pltpu.VMEM((EC, H), jnp.float32),
        pltpu.SemaphoreType.DMA,
    ],
)
def _dec_gather(tab_hbm, r_hbm, c_hbm, gr_hbm, gc_hbm, *rest):
    _dec_gather_body(tab_hbm, r_hbm, c_hbm, gr_hbm, gc_hbm, *rest)


BROW = 2000


def _linear_relu_body(x_ref, w_ref, b_ref, o_ref):
    y = _dot(x_ref[...], w_ref[...]) + b_ref[...]
    o_ref[...] = jnp.maximum(y, 0.0)


def _linear_relu(x, w, b):
    return pl.pallas_call(
        _linear_relu_body,
        grid=(N // BROW,),
        in_specs=[
            pl.BlockSpec((BROW, H), lambda i: (i, 0)),
            pl.BlockSpec((H, H), lambda i: (0, 0)),
            pl.BlockSpec((1, H), lambda i: (0, 0)),
        ],
        out_specs=pl.BlockSpec((BROW, H), lambda i: (i, 0)),
        out_shape=jax.ShapeDtypeStruct((N, H), jnp.float32),
    )(x, w, b.reshape(1, H))


def _wfold_body(wk_ref, bk_ref, arel_ref, wv_ref, bv_ref, mrel_ref, prel_ref,
                kf_ref, bkf_ref, vf_ref, bvf_ref):
    scale = prel_ref[0, 0] / jnp.sqrt(jnp.float32(H))
    kf_ref[...] = _dot(wk_ref[...], arel_ref[...]) * scale
    bkf_ref[...] = _dot(bk_ref[...], arel_ref[...]) * scale
    vf_ref[...] = _dot(wv_ref[...], mrel_ref[...])
    bvf_ref[...] = _dot(bv_ref[...], mrel_ref[...])


def _wfold(wk, bk, arel, wv, bv, mrel, prel):
    return pl.pallas_call(
        _wfold_body,
        out_shape=(jax.ShapeDtypeStruct((H, H), jnp.float32),
                   jax.ShapeDtypeStruct((1, H), jnp.float32),
                   jax.ShapeDtypeStruct((H, H), jnp.float32),
                   jax.ShapeDtypeStruct((1, H), jnp.float32)),
    )(wk, bk.reshape(1, H), arel, wv, bv.reshape(1, H), mrel,
      prel.reshape(1, 1))


def _qkv_body(x_ref, wq_ref, bq_ref, kf_ref, bkf_ref, vf_ref, bvf_ref,
              q_ref, k_ref, v_ref):
    x = x_ref[...]
    q_ref[...] = _dot(x, wq_ref[...]) + bq_ref[...]
    k_ref[...] = _dot(x, kf_ref[...]) + bkf_ref[...]
    v_ref[...] = _dot(x, vf_ref[...]) + bvf_ref[...]


def _qkv(x, wq, bq, kf, bkf, vf, bvf):
    full = lambda i: (0, 0)
    blk = lambda i: (i, 0)
    return pl.pallas_call(
        _qkv_body,
        grid=(N // BROW,),
        in_specs=[
            pl.BlockSpec((BROW, H), blk),
            pl.BlockSpec((H, H), full),
            pl.BlockSpec((1, H), full),
            pl.BlockSpec((H, H), full),
            pl.BlockSpec((1, H), full),
            pl.BlockSpec((H, H), full),
            pl.BlockSpec((1, H), full),
        ],
        out_specs=(pl.BlockSpec((BROW, H), blk),) * 3,
        out_shape=(jax.ShapeDtypeStruct((N, H), jnp.float32),) * 3,
    )(x, wq, bq.reshape(1, H), kf, bkf, vf, bvf)


def _post_a_body(acc_ref, xp_ref, wa_ref, ba_ref, skip_ref, y_ref, st_ref):
    i = pl.program_id(0)
    acc = acc_ref[0] + acc_ref[1]
    den = acc[:, H:H + 1]
    agg = acc[:, :H] / jnp.maximum(den, 1e-16)
    g = 0.5 * agg * (1.0 + lax.erf(agg / jnp.sqrt(jnp.float32(2.0))))
    out = _dot(g, wa_ref[...]) + ba_ref[...]
    s = jax.nn.sigmoid(skip_ref[0, 0])
    y = s * out + (1.0 - s) * xp_ref[...]
    y_ref[...] = y

    @pl.when(i == 0)
    def _():
        st_ref[...] = jnp.zeros_like(st_ref)

    st_ref[...] += jnp.concatenate(
        [jnp.sum(y, axis=0, keepdims=True),
         jnp.sum(y * y, axis=0, keepdims=True)], axis=0)


def _post_a(acc2, x_prev, wa, ba, skip):
    return pl.pallas_call(
        _post_a_body,
        grid=(N // BROW,),
        in_specs=[
            pl.BlockSpec((NC, BROW, AW), lambda i: (0, i, 0)),
            pl.BlockSpec((BROW, H), lambda i: (i, 0)),
            pl.BlockSpec((H, H), lambda i: (0, 0)),
            pl.BlockSpec((1, H), lambda i: (0, 0)),
            pl.BlockSpec((1, 1), lambda i: (0, 0)),
        ],
        out_specs=(pl.BlockSpec((BROW, H), lambda i: (i, 0)),
                   pl.BlockSpec((2, H), lambda i: (0, 0))),
        out_shape=(jax.ShapeDtypeStruct((N, H), jnp.float32),
                   jax.ShapeDtypeStruct((2, H), jnp.float32)),
    )(acc2, x_prev, wa, ba.reshape(1, H), skip.reshape(1, 1))


def _post_b_body(y_ref, st_ref, g_ref, b_ref, o_ref):
    mu = st_ref[0:1] / jnp.float32(N)
    var = st_ref[1:2] / jnp.float32(N) - mu * mu
    o_ref[...] = (y_ref[...] - mu) / jnp.sqrt(var + 1e-5) * g_ref[...] + b_ref[...]


def _post_b(y, st, gamma, beta):
    return pl.pallas_call(
        _post_b_body,
        grid=(N // BROW,),
        in_specs=[
            pl.BlockSpec((BROW, H), lambda i: (i, 0)),
            pl.BlockSpec((2, H), lambda i: (0, 0)),
            pl.BlockSpec((1, H), lambda i: (0, 0)),
            pl.BlockSpec((1, H), lambda i: (0, 0)),
        ],
        out_specs=pl.BlockSpec((BROW, H), lambda i: (i, 0)),
        out_shape=jax.ShapeDtypeStruct((N, H), jnp.float32),
    )(y, st, gamma.reshape(1, H), beta.reshape(1, H))


BDEC = 2048


def _dec_body(gr_ref, gc_ref, w1a_ref, w1b_ref, b1_ref, w2_ref, b2_ref, o_ref):
    z1 = _dot(gr_ref[...], w1a_ref[...]) + _dot(gc_ref[...], w1b_ref[...])
    z1 = jnp.maximum(z1 + b1_ref[...], 0.0)
    o_ref[...] = jnp.sum(z1 * w2_ref[...], axis=1) + b2_ref[0, 0]


def _decoder(gr, gc, w1a, w1b, b1, w2r, b2):
    full = lambda i: (0, 0)
    return pl.pallas_call(
        _dec_body,
        grid=(NPAD_D // BDEC,),
        in_specs=[
            pl.BlockSpec((BDEC, H), lambda i: (i, 0)),
            pl.BlockSpec((BDEC, H), lambda i: (i, 0)),
            pl.BlockSpec((H, H), full),
            pl.BlockSpec((H, H), full),
            pl.BlockSpec((1, H), full),
            pl.BlockSpec((1, H), full),
            pl.BlockSpec((1, 1), full),
        ],
        out_specs=pl.BlockSpec((BDEC,), lambda i: (i,)),
        out_shape=jax.ShapeDtypeStruct((NPAD_D,), jnp.float32),
    )(gr, gc, w1a, w1b, b1.reshape(1, H), w2r, b2.reshape(1, 1))


def _pad_tiles(idx, valid_per_tile, padded_per_tile):
    x = idx.astype(jnp.int32).reshape(NW, valid_per_tile)
    x = jnp.pad(x, ((0, 0), (0, padded_per_tile - valid_per_tile)))
    return x.reshape(-1)


def kernel(x_author, x_topic, edge_index_a2t, edge_index_t2a, edge_label_index, params):
    p = params

    rows_a2t = _pad_tiles(edge_index_a2t[0], EV_E, EPT_E)
    cols_a2t = _pad_tiles(edge_index_a2t[1], EV_E, EPT_E)
    rows_t2a = _pad_tiles(edge_index_t2a[0], EV_E, EPT_E)
    cols_t2a = _pad_tiles(edge_index_t2a[1], EV_E, EPT_E)
    dec_r = _pad_tiles(edge_label_index[0], NDEC // NW, EPT_D)
    dec_c = _pad_tiles(edge_label_index[1], NDEC // NW, EPT_D)

    xa = _linear_relu(x_author, p['W_in_author'], p['b_in_author'])
    xt = _linear_relu(x_topic, p['W_in_topic'], p['b_in_topic'])

    for l in range(2):
        last = l == 1
        kft, bkft, vft, bvft = _wfold(
            p[f'l{l}_Wk_topic'], p[f'l{l}_bk_topic'], p[f'l{l}_arel_t2a'],
            p[f'l{l}_Wv_topic'], p[f'l{l}_bv_topic'], p[f'l{l}_mrel_t2a'],
            p[f'l{l}_prel_t2a'])
        qa, ka, va = _qkv(xa, p[f'l{l}_Wq_author'], p[f'l{l}_bq_author'],
                          *( _wfold(
                              p[f'l{l}_Wk_author'], p[f'l{l}_bk_author'],
                              p[f'l{l}_arel_a2t'],
                              p[f'l{l}_Wv_author'], p[f'l{l}_bv_author'],
                              p[f'l{l}_mrel_a2t'], p[f'l{l}_prel_a2t'])))
        qt, kt, vt = _qkv(xt, p[f'l{l}_Wq_topic'], p[f'l{l}_bq_topic'],
                          kft, bkft, vft, bvft)

        if last:
            acc = _edge_agg1(kt, qa, vt, rows_t2a, cols_t2a)
            acc_a = acc[0]
        else:
            acc = _edge_agg2(ka, qt, va, rows_a2t, cols_a2t,
                             kt, qa, vt, rows_t2a, cols_t2a)
            acc_t, acc_a = acc[0], acc[1]

        ya, sta = _post_a(acc_a, xa, p[f'l{l}_Wa_author'], p[f'l{l}_ba_author'],
                          p[f'l{l}_skip_author'])
        xa = _post_b(ya, sta, p[f'l{l}_bn_gamma'], p[f'l{l}_bn_beta'])
        if not last:
            yt, stt = _post_a(acc_t, xt, p[f'l{l}_Wa_topic'], p[f'l{l}_ba_topic'],
                              p[f'l{l}_skip_topic'])
            xt = _post_b(yt, stt, p[f'l{l}_bn_gamma'], p[f'l{l}_bn_beta'])

    gr, gc = _dec_gather(xa, dec_r, dec_c)
    z = _decoder(gr, gc, p['dec_W1'][:H], p['dec_W1'][H:], p['dec_b1'],
                 p['dec_W2'].reshape(1, H), p['dec_b2'])
    return z.reshape(NW, EPT_D)[:, :NDEC // NW].reshape(-1)
